# baseline (speedup 1.0000x reference)
import jax
import jax.numpy as jnp
from jax.experimental import pallas as pl
from jax.experimental.pallas import tpu as pltpu

E = 8
K = 2
H = 1024
FF = 2880
S = 2048
T = 256
NT = 23
PMAX = 6144
FC = 768
NF = 4
LIMIT = 7.0
GROW = 2 * S


def _router_body(hid_ref, rw_ref, dest_ref, w_ref, cnt_ref, rank_ref):
    x = hid_ref[...]
    logits = jnp.dot(x, rw_ref[...], preferred_element_type=jnp.float32)
    eiota = jax.lax.broadcasted_iota(jnp.int32, (S, E), 1)
    m1 = jnp.max(logits, axis=1, keepdims=True)
    i1 = jnp.min(jnp.where(logits == m1, eiota, E), axis=1, keepdims=True)
    l2 = jnp.where(eiota == i1, -jnp.inf, logits)
    m2 = jnp.max(l2, axis=1, keepdims=True)
    i2 = jnp.min(jnp.where(l2 == m2, eiota, E), axis=1, keepdims=True)
    sexp = jnp.exp(m2 - m1)
    p1 = 1.0 / (1.0 + sexp)
    p2 = sexp / (1.0 + sexp)
    maskf = ((eiota == i1) | (eiota == i2)).astype(jnp.float32)
    cntf = jnp.sum(maskf, axis=0, keepdims=True)
    cnt_ref[...] = cntf.astype(jnp.int32)
    padded = jnp.ceil(cntf / T) * T
    r8 = jax.lax.broadcasted_iota(jnp.int32, (E, E), 0)
    c8 = jax.lax.broadcasted_iota(jnp.int32, (E, E), 1)
    tri = (r8 < c8).astype(jnp.float32)
    off = jnp.dot(padded, tri, preferred_element_type=jnp.float32)

    def body(b, _):
        r0 = b * T
        rowi = jax.lax.broadcasted_iota(jnp.int32, (T, S), 0) + r0
        coli = jax.lax.broadcasted_iota(jnp.int32, (T, S), 1)
        lb = (coli < rowi).astype(jnp.float32)
        rank_ref[pl.ds(r0, T), :] = jnp.dot(
            lb, maskf, preferred_element_type=jnp.float32)
        return 0

    jax.lax.fori_loop(0, S // T, body, 0)
    posf = off + rank_ref[...]
    sel1 = (eiota == i1).astype(jnp.float32)
    sel2 = (eiota == i2).astype(jnp.float32)
    d1 = jnp.sum(sel1 * posf, axis=1, keepdims=True)
    d2 = jnp.sum(sel2 * posf, axis=1, keepdims=True)
    kiota = jax.lax.broadcasted_iota(jnp.int32, (S, K), 1)
    dest_ref[...] = jnp.where(kiota == 0, d1, d2).astype(jnp.int32)
    w_ref[...] = jnp.where(kiota == 0, p1, p2)


def _router(hid, rw):
    return pl.pallas_call(
        _router_body,
        out_shape=[
            jax.ShapeDtypeStruct((S, K), jnp.int32),
            jax.ShapeDtypeStruct((S, K), jnp.float32),
            jax.ShapeDtypeStruct((1, E), jnp.int32),
        ],
        scratch_shapes=[pltpu.VMEM((S, E), jnp.float32)],
    )(hid, rw)


def _moe_body(meta_ref, x_ref, g_ref, u_ref, d_ref, w_ref, o_ref):
    t = pl.program_id(0)
    f = pl.program_id(1)
    nt = meta_ref[0]

    @pl.when(t < nt)
    def _():
        x = x_ref[...]
        g = jnp.dot(x, g_ref[0], preferred_element_type=jnp.float32)
        g = g * jax.nn.sigmoid(g)
        g = jnp.clip(g, -LIMIT, LIMIT)
        u = jnp.dot(x, u_ref[0], preferred_element_type=jnp.float32)
        ff0 = f * FC
        hcol = jax.lax.broadcasted_iota(jnp.int32, (T, FC), 1) + ff0
        h = jnp.where(hcol < FF, g * u, 0.0)
        drow = jax.lax.broadcasted_iota(jnp.int32, (FC, H), 0) + ff0
        d = jnp.where(drow < FF, d_ref[0], 0.0)
        y = jnp.dot(h, d, preferred_element_type=jnp.float32)

        @pl.when(f == 0)
        def _():
            o_ref[...] = y

        @pl.when(f > 0)
        def _():
            o_ref[...] = o_ref[...] + y

        @pl.when(f == NF - 1)
        def _():
            o_ref[...] = o_ref[...] * w_ref[...]


def _moe(meta, xs, gate_w, up_w, down_w, wsort):
    grid_spec = pltpu.PrefetchScalarGridSpec(
        num_scalar_prefetch=1,
        grid=(NT, NF),
        in_specs=[
            pl.BlockSpec((T, H), lambda t, f, m: (t, 0)),
            pl.BlockSpec((1, H, FC), lambda t, f, m: (m[1 + t], 0, f)),
            pl.BlockSpec((1, H, FC), lambda t, f, m: (m[1 + t], 0, f)),
            pl.BlockSpec((1, FC, H), lambda t, f, m: (m[1 + t], f, 0)),
            pl.BlockSpec((T, 1), lambda t, f, m: (t, 0)),
        ],
        out_specs=pl.BlockSpec((T, H), lambda t, f, m: (t, 0)),
    )
    return pl.pallas_call(
        _moe_body,
        grid_spec=grid_spec,
        out_shape=jax.ShapeDtypeStruct((PMAX, H), jnp.float32),
    )(meta, xs, gate_w, up_w, down_w, wsort)


def kernel(hidden_states, router_weights, gate_w, up_w, down_w):
    hid = hidden_states.reshape(S, H)
    dest, w, cnt = _router(hid, router_weights)
    cnt = cnt.reshape(E)
    tiles_per = (cnt + (T - 1)) // T
    ntiles = jnp.sum(tiles_per).astype(jnp.int32)
    tile_ex = jnp.repeat(
        jnp.arange(E, dtype=jnp.int32), tiles_per, total_repeat_length=NT)
    meta = jnp.concatenate([ntiles.reshape(1), tile_ex])

    destf = dest.reshape(S * K)
    j = jnp.arange(S * K, dtype=jnp.int32)
    payload = (j & 1) * S + (j >> 1)
    destrow = jnp.full((PMAX,), GROW, jnp.int32).at[destf].set(payload)
    wsort = jnp.zeros((PMAX, 1), jnp.float32).at[destf, 0].set(w.reshape(S * K))
    gidx = destrow & (S - 1)
    xs = hid[gidx]

    y = _moe(meta, xs, gate_w, up_w, down_w, wsort)

    buf = jnp.zeros((2 * S + 8, H), jnp.float32).at[destrow].set(y)
    out = buf[:S] + buf[S:2 * S]
    return out.reshape(1, S, H)
